# baseline (speedup 1.0000x reference)
import functools

import jax
import jax.numpy as jnp
from jax import lax
from jax.experimental import pallas as pl
from jax.experimental.pallas import tpu as pltpu
from jax.experimental.pallas import tpu_sc as plsc

NTOKENS = 100000
NHID = 128
NSAMPLED = 1024
BATCH = 1024
NROWS = BATCH + NSAMPLED


def _sc_gather(weight, labels, sample_ids):
    info = plsc.get_sparse_core_info()
    ns = info.num_subcores
    b_per_w = BATCH // ns
    mesh = plsc.VectorSubcoreMesh(core_axis_name="c", subcore_axis_name="s",
                                  num_cores=1)

    @functools.partial(
        pl.kernel,
        mesh=mesh,
        out_type=(
            jax.ShapeDtypeStruct((BATCH, NHID), jnp.float32),
            jax.ShapeDtypeStruct((NSAMPLED, NHID), jnp.float32),
        ),
        scratch_types=[
            pltpu.VMEM((b_per_w,), jnp.int32),
            pltpu.VMEM((b_per_w,), jnp.int32),
            pltpu.VMEM((b_per_w, NHID), jnp.float32),
            pltpu.VMEM((b_per_w, NHID), jnp.float32),
            pltpu.SemaphoreType.DMA,
            pltpu.SemaphoreType.DMA,
            pltpu.SemaphoreType.DMA,
            pltpu.SemaphoreType.DMA,
        ],
    )
    def gather_kernel(table_hbm, labels_hbm, samples_hbm, tw_hbm, sw_hbm,
                      idx_a, idx_b, rows_a, rows_b, sem_a, sem_b,
                      sem_wa, sem_wb):
        w = lax.axis_index("s")
        base = w * b_per_w
        ia = pltpu.async_copy(labels_hbm.at[pl.ds(base, b_per_w)], idx_a, sem_a)
        ib = pltpu.async_copy(samples_hbm.at[pl.ds(base, b_per_w)], idx_b, sem_b)
        ia.wait()
        ga = pltpu.async_copy(table_hbm.at[idx_a], rows_a, sem_a)
        ib.wait()
        gb = pltpu.async_copy(table_hbm.at[idx_b], rows_b, sem_b)
        ga.wait()
        wa = pltpu.async_copy(rows_a, tw_hbm.at[pl.ds(base, b_per_w)], sem_wa)
        gb.wait()
        wb = pltpu.async_copy(rows_b, sw_hbm.at[pl.ds(base, b_per_w)], sem_wb)
        wa.wait()
        wb.wait()

    return gather_kernel(weight, labels, sample_ids)


def _colsum(mat, n):
    ones = jnp.ones((n, 1), jnp.float32)
    return lax.dot_general(mat, ones, (((1,), (0,)), ((), ())),
                           preferred_element_type=jnp.float32)


def _rowsum_t(mat, n):
    ones = jnp.ones((1, n), jnp.float32)
    return lax.dot_general(ones, mat, (((1,), (1,)), ((), ())),
                           preferred_element_type=jnp.float32)


def _tca_body(x_ref, aaug_ref):
    x = x_ref[...]
    xn = _colsum(x * x, NHID)
    ones_b = jnp.ones((BATCH, 1), jnp.float32)
    aaug_ref[...] = jnp.concatenate([x, xn, ones_b], axis=1)


def _tc_body(aaug_ref, tw_ref, sw_ref, out_ref):
    a_aug = aaug_ref[...]
    x = a_aug[:, 0:NHID]
    tw = tw_ref[...]
    sw = sw_ref[...]

    d = x - tw
    tn2 = _rowsum_t(d * d, NHID)
    sn = _colsum(sw * sw, NHID)

    ones_s = jnp.ones((NSAMPLED, 1), jnp.float32)
    c_aug = jnp.concatenate([-2.0 * sw, ones_s, sn], axis=1)
    dist2 = lax.dot_general(a_aug, c_aug, (((1,), (1,)), ((), ())),
                            preferred_element_type=jnp.float32)
    m = jnp.maximum(dist2, 1e-30)
    e = jnp.exp(m * lax.rsqrt(m))
    s = _rowsum_t(e, NSAMPLED)
    res = jnp.sqrt(tn2) - jnp.log(s)
    out_ref[...] = res.reshape(BATCH)


def kernel(inputs, labels, sample_ids, weight):
    a_aug = pl.pallas_call(
        _tca_body,
        out_shape=jax.ShapeDtypeStruct((BATCH, NHID + 2), jnp.float32),
    )(inputs)

    tw, sw = _sc_gather(weight, labels.astype(jnp.int32),
                        sample_ids.astype(jnp.int32))

    return pl.pallas_call(
        _tc_body,
        out_shape=jax.ShapeDtypeStruct((BATCH,), jnp.float32),
    )(a_aug, tw, sw)


# speedup vs baseline: 1.0313x; 1.0313x over previous
import functools

import jax
import jax.numpy as jnp
from jax import lax
from jax.experimental import pallas as pl
from jax.experimental.pallas import tpu as pltpu
from jax.experimental.pallas import tpu_sc as plsc

NTOKENS = 100000
NHID = 128
NSAMPLED = 1024
BATCH = 1024
NROWS = BATCH + NSAMPLED


def _sc_gather(weight, labels, sample_ids):
    info = plsc.get_sparse_core_info()
    ns = info.num_subcores
    b_per_w = BATCH // ns
    mesh = plsc.VectorSubcoreMesh(core_axis_name="c", subcore_axis_name="s",
                                  num_cores=1)

    @functools.partial(
        pl.kernel,
        mesh=mesh,
        out_type=(
            jax.ShapeDtypeStruct((BATCH, NHID), jnp.float32),
            jax.ShapeDtypeStruct((NSAMPLED, NHID), jnp.float32),
        ),
        scratch_types=[
            pltpu.VMEM((b_per_w,), jnp.int32),
            pltpu.VMEM((b_per_w,), jnp.int32),
            pltpu.VMEM((b_per_w, NHID), jnp.float32),
            pltpu.VMEM((b_per_w, NHID), jnp.float32),
            pltpu.SemaphoreType.DMA,
            pltpu.SemaphoreType.DMA,
            pltpu.SemaphoreType.DMA,
            pltpu.SemaphoreType.DMA,
        ],
    )
    def gather_kernel(table_hbm, labels_hbm, samples_hbm, tw_hbm, sw_hbm,
                      idx_a, idx_b, rows_a, rows_b, sem_a, sem_b,
                      sem_wa, sem_wb):
        w = lax.axis_index("s")
        base = w * b_per_w
        ia = pltpu.async_copy(labels_hbm.at[pl.ds(base, b_per_w)], idx_a, sem_a)
        ib = pltpu.async_copy(samples_hbm.at[pl.ds(base, b_per_w)], idx_b, sem_b)
        ia.wait()
        ga = pltpu.async_copy(table_hbm.at[idx_a], rows_a, sem_a)
        ib.wait()
        gb = pltpu.async_copy(table_hbm.at[idx_b], rows_b, sem_b)
        ga.wait()
        wa = pltpu.async_copy(rows_a, tw_hbm.at[pl.ds(base, b_per_w)], sem_wa)
        gb.wait()
        wb = pltpu.async_copy(rows_b, sw_hbm.at[pl.ds(base, b_per_w)], sem_wb)
        wa.wait()
        wb.wait()

    return gather_kernel(weight, labels, sample_ids)


def _colsum(mat, n):
    ones = jnp.ones((n, 1), jnp.float32)
    return lax.dot_general(mat, ones, (((1,), (0,)), ((), ())),
                           preferred_element_type=jnp.float32)


def _rowsum_t(mat, n):
    ones = jnp.ones((1, n), jnp.float32)
    return lax.dot_general(ones, mat, (((1,), (1,)), ((), ())),
                           preferred_element_type=jnp.float32)


def _tc_body(x_ref, tw_ref, sw_ref, out_ref):
    x = x_ref[...]
    tw = tw_ref[...]
    sw = sw_ref[...]

    d = x - tw
    tn2 = _rowsum_t(d * d, NHID)
    xn = _colsum(x * x, NHID)
    sn = _colsum(sw * sw, NHID)

    ones_b = jnp.ones((BATCH, 1), jnp.float32)
    ones_s = jnp.ones((NSAMPLED, 1), jnp.float32)
    a_aug = jnp.concatenate([x, xn, ones_b], axis=1)
    c_aug = jnp.concatenate([-2.0 * sw, ones_s, sn], axis=1)
    dist2 = lax.dot_general(a_aug, c_aug, (((1,), (1,)), ((), ())),
                            preferred_element_type=jnp.float32)
    m = jnp.maximum(dist2, 1e-30)
    e = jnp.exp(m * lax.rsqrt(m))
    s = _rowsum_t(e, NSAMPLED)
    res = jnp.sqrt(tn2) - jnp.log(s)
    out_ref[...] = res.reshape(BATCH)


def kernel(inputs, labels, sample_ids, weight):
    tw, sw = _sc_gather(weight, labels.astype(jnp.int32),
                        sample_ids.astype(jnp.int32))

    return pl.pallas_call(
        _tc_body,
        out_shape=jax.ShapeDtypeStruct((BATCH,), jnp.float32),
    )(inputs, tw, sw)
